# baseline (speedup 1.0000x reference)
import functools

import jax
import jax.numpy as jnp
from jax import lax
from jax.experimental import pallas as pl
from jax.experimental.pallas import tpu as pltpu
from jax.experimental.pallas import tpu_sc as plsc
from jax.scipy.linalg import block_diag

B, L, D = 512, 48, 6
AF, BF, FP = 39, 10, 64
APAD, BPAD = 48, 16
SLOT = APAD + BPAD
CW = D * SLOT
NBW = D * FP
T = 2

NC, NS = 2, 16
NW = NC * NS
ROWS = B * L * D
RPW = ROWS // NW
CHUNK = 1536
NCHUNK = RPW // CHUNK
CATOM = CHUNK // D

BM = 32
GRID = B // BM
R = BM * L


def _sc_gather(atom_tab, bond_tab, gia_t, gib_t):
  mesh = plsc.VectorSubcoreMesh(core_axis_name="c", subcore_axis_name="s")

  @functools.partial(
      pl.kernel,
      mesh=mesh,
      out_type=jax.ShapeDtypeStruct((B * L, CW), jnp.bfloat16),
      scratch_types=[
          pltpu.VMEM((2, CHUNK), jnp.int32),
          pltpu.VMEM((2, CHUNK), jnp.int32),
          pltpu.VMEM((2, CHUNK, APAD), jnp.bfloat16),
          pltpu.VMEM((2, CHUNK, BPAD), jnp.bfloat16),
          pltpu.SemaphoreType.DMA,
          pltpu.SemaphoreType.DMA,
      ],
      compiler_params=pltpu.CompilerParams(use_tc_tiling_on_sc=False),
  )
  def k(atom_hbm, bond_hbm, ia_hbm, ib_hbm, cnb_hbm,
        ia_v, ib_v, a_v, b_v, sem_g, sem_w):
    wid = lax.axis_index("s") * NC + lax.axis_index("c")

    def start(c):
      p = c % 2
      off = (wid * NCHUNK + c) * CHUNK
      pltpu.sync_copy(ia_hbm.at[pl.ds(off, CHUNK)], ia_v.at[p])
      pltpu.sync_copy(ib_hbm.at[pl.ds(off, CHUNK)], ib_v.at[p])
      return (pltpu.async_copy(atom_hbm.at[ia_v.at[p]], a_v.at[p], sem_g),
              pltpu.async_copy(bond_hbm.at[ib_v.at[p]], b_v.at[p], sem_g))

    pending_w = {0: [], 1: []}
    pending_g = {0: None, 1: None}
    pending_g[0] = start(0)
    for c in range(NCHUNK):
      p = c % 2
      q = (c + 1) % 2
      if c + 1 < NCHUNK:
        for cp in pending_w[q]:
          cp.wait()
        pending_w[q] = []
        pending_g[q] = start(c + 1)
      for cp in pending_g[p]:
        cp.wait()
      aoff = (wid * NCHUNK + c) * CATOM
      ws = []
      for d in range(D):
        ws.append(pltpu.async_copy(
            a_v.at[p, pl.ds(d * CATOM, CATOM)],
            cnb_hbm.at[pl.ds(aoff, CATOM), pl.ds(d * SLOT, APAD)], sem_w))
        ws.append(pltpu.async_copy(
            b_v.at[p, pl.ds(d * CATOM, CATOM)],
            cnb_hbm.at[pl.ds(aoff, CATOM), pl.ds(d * SLOT + APAD, BPAD)], sem_w))
      pending_w[p] = ws
    for p in (0, 1):
      for cp in pending_w[p]:
        cp.wait()

  return k(atom_tab, bond_tab, gia_t, gib_t)


def _mm(a, b):
  return lax.dot_general(a, b, (((1,), (0,)), ((), ())),
                         preferred_element_type=jnp.float32)


def _lrelu(x):
  return jnp.where(x >= 0, x, 0.01 * x)


def _elu(x):
  return jnp.where(x > 0, x, jnp.exp(jnp.minimum(x, 0.0)) - 1.0)


def _gru(x, h, wih, whh, bih, bhh):
  g_r = _mm(x, wih[0]) + bih[0] + _mm(h, whh[0]) + bhh[0]
  g_z = _mm(x, wih[1]) + bih[1] + _mm(h, whh[1]) + bhh[1]
  i_n = _mm(x, wih[2]) + bih[2]
  h_n = _mm(h, whh[2]) + bhh[2]
  r = jax.nn.sigmoid(g_r)
  z = jax.nn.sigmoid(g_z)
  n = jnp.tanh(i_n + r * h_n)
  return (1.0 - z) * n + z * h


def _tc_body(atom_ref, cnb_ref, adl_ref, amask_ref,
             wfc_ref, bfc_ref, wcblk_ref, bnbb_ref,
             w1a_ref, w2blk_ref, balign_ref,
             e6_ref, a0stk_ref, a0b_ref, a1t_ref, a1b_ref,
             wih_ref, whh_ref, bih_ref, bhh_ref,
             w1m_ref, w2m_ref, bmal_ref, amt_ref, amb_ref,
             mwih_ref, mwhh_ref, mbih_ref, mbhh_ref,
             outw_ref, outb_ref,
             af_out, pred_out):
  af = _lrelu(_mm(atom_ref[...], wfc_ref[...]) + bfc_ref[...])

  nbf = _lrelu(_mm(cnb_ref[...], wcblk_ref[...]) + bnbb_ref[...])

  adl = adl_ref[...]
  is_pad = adl == (L - 1)
  att = jnp.where(is_pad, 0.0, 1.0)
  smask = jnp.where(is_pad, -9e8, 0.0)
  cnt = jnp.sum(att, axis=1, keepdims=True)
  has = jnp.where(cnt > 0.0, 1.0, 0.0)

  u = _mm(af, w1a_ref[...]) + balign_ref[0, 0]
  v = _mm(nbf, w2blk_ref[...])
  s = _lrelu(u + v) + smask
  e = jnp.exp(s - jnp.max(s, axis=1, keepdims=True))
  aw = e / jnp.sum(e, axis=1, keepdims=True) * att
  awx = _mm(aw, e6_ref[...])
  ctx = _elu(_mm(nbf * awx, a0stk_ref[...]) + has * a0b_ref[...])
  af = _gru(ctx, af,
            (wih_ref[0, 0], wih_ref[0, 1], wih_ref[0, 2]),
            (whh_ref[0, 0], whh_ref[0, 1], whh_ref[0, 2]),
            (bih_ref[0, 0], bih_ref[0, 1], bih_ref[0, 2]),
            (bhh_ref[0, 0], bhh_ref[0, 1], bhh_ref[0, 2]))

  rfeat = jnp.maximum(af, 0.0)
  ctx1 = _elu(has * (_mm(rfeat, a1t_ref[...]) + a1b_ref[...]))
  af = _gru(ctx1, af,
            (wih_ref[1, 0], wih_ref[1, 1], wih_ref[1, 2]),
            (whh_ref[1, 0], whh_ref[1, 1], whh_ref[1, 2]),
            (bih_ref[1, 0], bih_ref[1, 1], bih_ref[1, 2]),
            (bhh_ref[1, 0], bhh_ref[1, 1], bhh_ref[1, 2]))
  af_out[...] = af

  amask = amask_ref[...]
  molf = jnp.sum((jnp.maximum(af, 0.0) * amask).reshape(BM, L, FP), axis=1)
  molsm = jnp.where(amask == 0.0, -9e8, 0.0).reshape(BM, L, 1)
  amask3 = amask.reshape(BM, L, 1)
  hasm = jnp.where(jnp.sum(amask3, axis=1) > 0.0, 1.0, 0.0)
  mwih = (mwih_ref[0], mwih_ref[1], mwih_ref[2])
  mwhh = (mwhh_ref[0], mwhh_ref[1], mwhh_ref[2])
  mbih = (mbih_ref[0], mbih_ref[1], mbih_ref[2])
  mbhh = (mbhh_ref[0], mbhh_ref[1], mbhh_ref[2])
  af3 = af.reshape(BM, L, FP)
  vm = (_mm(af, w2m_ref[...]) + bmal_ref[0, 0]).reshape(BM, L, 1)
  for _ in range(T):
    um = _mm(molf, w1m_ref[...])
    sm = _lrelu(jnp.broadcast_to(um.reshape(BM, 1, 1), (BM, L, 1)) + vm) + molsm
    em = jnp.exp(sm - jnp.max(sm, axis=1, keepdims=True))
    mw = em / jnp.sum(em, axis=1, keepdims=True) * amask3
    afw = jnp.sum(af3 * mw, axis=1)
    mc = _elu(_mm(afw, amt_ref[...]) + hasm * amb_ref[...])
    molf = _gru(mc, molf, mwih, mwhh, mbih, mbhh)
  pred_out[...] = _mm(molf, outw_ref[...]) + outb_ref[0, 0]


def _tc_specs():
  full = lambda shape: pl.BlockSpec(shape, lambda i, _s=len(shape): (0,) * _s)
  in_specs = [
      pl.BlockSpec((R, AF), lambda i: (i, 0)),
      pl.BlockSpec((R, CW), lambda i: (i, 0)),
      pl.BlockSpec((R, D), lambda i: (i, 0)),
      pl.BlockSpec((R, 1), lambda i: (i, 0)),
      full((AF, FP)), full((1, FP)),
      full((CW, NBW)), full((1, NBW)),
      full((FP, 1)), full((NBW, D)), full((1, 1)),
      full((D, NBW)),
      full((NBW, FP)), full((1, FP)),
      full((FP, FP)), full((1, FP)),
      full((2, 3, FP, FP)), full((2, 3, FP, FP)),
      full((2, 3, 1, FP)), full((2, 3, 1, FP)),
      full((FP, 1)), full((FP, 1)), full((1, 1)),
      full((FP, FP)), full((1, FP)),
      full((3, FP, FP)), full((3, FP, FP)),
      full((3, 1, FP)), full((3, 1, FP)),
      full((FP, 1)), full((1, 1)),
  ]
  out_specs = [
      pl.BlockSpec((R, FP), lambda i: (i, 0)),
      pl.BlockSpec((BM, 1), lambda i: (i, 0)),
  ]
  out_shape = [
      jax.ShapeDtypeStruct((B * L, FP), jnp.float32),
      jax.ShapeDtypeStruct((B, 1), jnp.float32),
  ]
  return (GRID,), in_specs, out_specs, out_shape


def _prep_params(P):
  f32 = lambda x: x.astype(jnp.float32)
  nbW = f32(P['nb_fc_W'])
  wa_t = jnp.pad(nbW[:, :AF].T, ((0, APAD - AF), (0, 0)))
  wb_t = jnp.pad(nbW[:, AF:].T, ((0, BPAD - BF), (0, 0)))
  w2a = f32(P['align_W'])[0, :, FP:].reshape(FP, 1)
  wih = f32(P['gru_Wih']).reshape(2, 3, FP, FP).transpose(0, 1, 3, 2)
  whh = f32(P['gru_Whh']).reshape(2, 3, FP, FP).transpose(0, 1, 3, 2)
  mwih = f32(P['mol_gru_Wih']).reshape(3, FP, FP).transpose(0, 2, 1)
  mwhh = f32(P['mol_gru_Whh']).reshape(3, FP, FP).transpose(0, 2, 1)
  wc = jnp.concatenate([wa_t, wb_t], axis=0)
  return [
      f32(P['atom_fc_W']).T, f32(P['atom_fc_b']).reshape(1, FP),
      block_diag(*([wc] * D)).astype(jnp.bfloat16),
      jnp.tile(f32(P['nb_fc_b']).reshape(1, FP), (1, D)),
      f32(P['align_W'])[0, :, :FP].reshape(FP, 1),
      block_diag(*([w2a] * D)),
      f32(P['align_b'])[0].reshape(1, 1),
      jnp.kron(jnp.eye(D, dtype=jnp.float32), jnp.ones((1, FP), jnp.float32)),
      jnp.tile(f32(P['attend_W'])[0].T, (D, 1)),
      f32(P['attend_b'])[0].reshape(1, FP),
      f32(P['attend_W'])[1].T, f32(P['attend_b'])[1].reshape(1, FP),
      wih, whh,
      f32(P['gru_bih']).reshape(2, 3, 1, FP), f32(P['gru_bhh']).reshape(2, 3, 1, FP),
      f32(P['mol_align_W'])[:, :FP].reshape(FP, 1),
      f32(P['mol_align_W'])[:, FP:].reshape(FP, 1),
      f32(P['mol_align_b']).reshape(1, 1),
      f32(P['mol_attend_W']).T, f32(P['mol_attend_b']).reshape(1, FP),
      mwih, mwhh,
      f32(P['mol_gru_bih']).reshape(3, 1, FP), f32(P['mol_gru_bhh']).reshape(3, 1, FP),
      f32(P['out_W']).reshape(FP, 1), f32(P['out_b']).reshape(1, 1),
  ]


def kernel(atom_list, bond_list, atom_degree_list, bond_degree_list,
           atom_mask, params):
  atom2 = atom_list.reshape(B * L, AF).astype(jnp.float32)
  atom_tab = jnp.pad(atom2, ((0, 0), (0, APAD - AF))).astype(jnp.bfloat16)
  bond_tab = jnp.pad(bond_list.reshape(B * L, BF).astype(jnp.float32),
                     ((0, 0), (0, BPAD - BF))).astype(jnp.bfloat16)
  adl = atom_degree_list.astype(jnp.int32)
  bdl = bond_degree_list.astype(jnp.int32)
  base = (jnp.arange(B, dtype=jnp.int32) * L)[:, None, None]
  ngrp = B * L // CATOM
  gia_t = (adl + base).reshape(ngrp, CATOM, D).transpose(0, 2, 1).reshape(ROWS)
  gib_t = (bdl + base).reshape(ngrp, CATOM, D).transpose(0, 2, 1).reshape(ROWS)

  cnb = _sc_gather(atom_tab, bond_tab, gia_t, gib_t)

  grid, in_specs, out_specs, out_shape = _tc_specs()
  af2, pred = pl.pallas_call(
      _tc_body,
      grid=grid,
      in_specs=in_specs,
      out_specs=out_specs,
      out_shape=out_shape,
      compiler_params=pltpu.CompilerParams(
          dimension_semantics=("arbitrary",),
          vmem_limit_bytes=100 * 1024 * 1024,
      ),
  )(atom2, cnb, adl.reshape(B * L, D),
    atom_mask.reshape(B * L, 1).astype(jnp.float32),
    *_prep_params(params))
  return af2.reshape(B, L, FP), pred


# speedup vs baseline: 1.2571x; 1.2571x over previous
import functools

import jax
import jax.numpy as jnp
from jax import lax
from jax.experimental import pallas as pl
from jax.experimental.pallas import tpu as pltpu
from jax.experimental.pallas import tpu_sc as plsc
from jax.scipy.linalg import block_diag

B, L, D = 512, 48, 6
AF, BF, FP = 39, 10, 64
APAD, BPAD = 48, 16
SLOT = APAD + BPAD
CW = D * SLOT
NBW = D * FP
T = 2

NC, NS = 2, 16
NW = NC * NS
ROWS = B * L * D
RPW = ROWS // NW
CHUNK = 768
NCHUNK = RPW // CHUNK
CATOM = CHUNK // D
NPC = CW // 128

BM = 32
GRID = B // BM
R = BM * L


def _sc_gather(atom_tab, bond_tab, gia_t, gib_t):
  mesh = plsc.VectorSubcoreMesh(core_axis_name="c", subcore_axis_name="s")

  @functools.partial(
      pl.kernel,
      mesh=mesh,
      out_type=jax.ShapeDtypeStruct((NPC, B * L, 128), jnp.float32),
      scratch_types=[
          pltpu.VMEM((2, CHUNK), jnp.int32),
          pltpu.VMEM((2, CHUNK), jnp.int32),
          pltpu.VMEM((2, CHUNK, APAD), jnp.float32),
          pltpu.VMEM((2, CHUNK, BPAD), jnp.float32),
          pltpu.SemaphoreType.DMA,
          pltpu.SemaphoreType.DMA,
      ],
      compiler_params=pltpu.CompilerParams(use_tc_tiling_on_sc=False),
  )
  def k(atom_hbm, bond_hbm, ia_hbm, ib_hbm, cnb_hbm,
        ia_v, ib_v, a_v, b_v, sem_g, sem_w):
    wid = lax.axis_index("s") * NC + lax.axis_index("c")

    def start(c):
      p = c % 2
      off = (wid * NCHUNK + c) * CHUNK
      pltpu.sync_copy(ia_hbm.at[pl.ds(off, CHUNK)], ia_v.at[p])
      pltpu.sync_copy(ib_hbm.at[pl.ds(off, CHUNK)], ib_v.at[p])
      return (pltpu.async_copy(atom_hbm.at[ia_v.at[p]], a_v.at[p], sem_g),
              pltpu.async_copy(bond_hbm.at[ib_v.at[p]], b_v.at[p], sem_g))

    pending_w = {0: [], 1: []}
    pending_g = {0: None, 1: None}
    pending_g[0] = start(0)
    for c in range(NCHUNK):
      p = c % 2
      q = (c + 1) % 2
      if c + 1 < NCHUNK:
        for cp in pending_w[q]:
          cp.wait()
        pending_w[q] = []
        pending_g[q] = start(c + 1)
      for cp in pending_g[p]:
        cp.wait()
      aoff = (wid * NCHUNK + c) * CATOM
      ws = []
      for d in range(D):
        lane = (d % 2) * SLOT
        ws.append(pltpu.async_copy(
            a_v.at[p, pl.ds(d * CATOM, CATOM)],
            cnb_hbm.at[d // 2, pl.ds(aoff, CATOM), pl.ds(lane, APAD)], sem_w))
        ws.append(pltpu.async_copy(
            b_v.at[p, pl.ds(d * CATOM, CATOM)],
            cnb_hbm.at[d // 2, pl.ds(aoff, CATOM), pl.ds(lane + APAD, BPAD)],
            sem_w))
      pending_w[p] = ws
    for p in (0, 1):
      for cp in pending_w[p]:
        cp.wait()

  return k(atom_tab, bond_tab, gia_t, gib_t)


def _mm(a, b):
  return lax.dot_general(a, b, (((1,), (0,)), ((), ())),
                         preferred_element_type=jnp.float32)


def _lrelu(x):
  return jnp.where(x >= 0, x, 0.01 * x)


def _elu(x):
  return jnp.where(x > 0, x, jnp.exp(jnp.minimum(x, 0.0)) - 1.0)


def _gru(x, h, wih, whh, bih, bhh):
  g_r = _mm(x, wih[0]) + bih[0] + _mm(h, whh[0]) + bhh[0]
  g_z = _mm(x, wih[1]) + bih[1] + _mm(h, whh[1]) + bhh[1]
  i_n = _mm(x, wih[2]) + bih[2]
  h_n = _mm(h, whh[2]) + bhh[2]
  r = jax.nn.sigmoid(g_r)
  z = jax.nn.sigmoid(g_z)
  n = jnp.tanh(i_n + r * h_n)
  return (1.0 - z) * n + z * h


def _tc_body(atom_ref, cnb_ref, adl_ref, amask_ref,
             wfc_ref, bfc_ref, wcblk_ref, bnbb_ref,
             w1a_ref, w2blk_ref, balign_ref,
             e6_ref, a0stk_ref, a0b_ref, a1t_ref, a1b_ref,
             wih_ref, whh_ref, bih_ref, bhh_ref,
             w1m_ref, w2m_ref, bmal_ref, amt_ref, amb_ref,
             mwih_ref, mwhh_ref, mbih_ref, mbhh_ref,
             outw_ref, outb_ref,
             af_out, pred_out):
  af = _lrelu(_mm(atom_ref[...], wfc_ref[...]) + bfc_ref[...])

  nbf = _lrelu(_mm(cnb_ref[0], wcblk_ref[0]) +
               _mm(cnb_ref[1], wcblk_ref[1]) +
               _mm(cnb_ref[2], wcblk_ref[2]) + bnbb_ref[...])

  adl = adl_ref[...]
  is_pad = adl == (L - 1)
  att = jnp.where(is_pad, 0.0, 1.0)
  smask = jnp.where(is_pad, -9e8, 0.0)
  cnt = jnp.sum(att, axis=1, keepdims=True)
  has = jnp.where(cnt > 0.0, 1.0, 0.0)

  u = _mm(af, w1a_ref[...]) + balign_ref[0, 0]
  v = _mm(nbf, w2blk_ref[...])
  s = _lrelu(u + v) + smask
  e = jnp.exp(s - jnp.max(s, axis=1, keepdims=True))
  aw = e / jnp.sum(e, axis=1, keepdims=True) * att
  awx = _mm(aw, e6_ref[...])
  ctx = _elu(_mm(nbf * awx, a0stk_ref[...]) + has * a0b_ref[...])
  af = _gru(ctx, af,
            (wih_ref[0, 0], wih_ref[0, 1], wih_ref[0, 2]),
            (whh_ref[0, 0], whh_ref[0, 1], whh_ref[0, 2]),
            (bih_ref[0, 0], bih_ref[0, 1], bih_ref[0, 2]),
            (bhh_ref[0, 0], bhh_ref[0, 1], bhh_ref[0, 2]))

  rfeat = jnp.maximum(af, 0.0)
  ctx1 = _elu(has * (_mm(rfeat, a1t_ref[...]) + a1b_ref[...]))
  af = _gru(ctx1, af,
            (wih_ref[1, 0], wih_ref[1, 1], wih_ref[1, 2]),
            (whh_ref[1, 0], whh_ref[1, 1], whh_ref[1, 2]),
            (bih_ref[1, 0], bih_ref[1, 1], bih_ref[1, 2]),
            (bhh_ref[1, 0], bhh_ref[1, 1], bhh_ref[1, 2]))
  af_out[...] = af

  amask = amask_ref[...]
  molf = jnp.sum((jnp.maximum(af, 0.0) * amask).reshape(BM, L, FP), axis=1)
  molsm = jnp.where(amask == 0.0, -9e8, 0.0).reshape(BM, L, 1)
  amask3 = amask.reshape(BM, L, 1)
  hasm = jnp.where(jnp.sum(amask3, axis=1) > 0.0, 1.0, 0.0)
  mwih = (mwih_ref[0], mwih_ref[1], mwih_ref[2])
  mwhh = (mwhh_ref[0], mwhh_ref[1], mwhh_ref[2])
  mbih = (mbih_ref[0], mbih_ref[1], mbih_ref[2])
  mbhh = (mbhh_ref[0], mbhh_ref[1], mbhh_ref[2])
  af3 = af.reshape(BM, L, FP)
  vm = (_mm(af, w2m_ref[...]) + bmal_ref[0, 0]).reshape(BM, L, 1)
  for _ in range(T):
    um = _mm(molf, w1m_ref[...])
    sm = _lrelu(jnp.broadcast_to(um.reshape(BM, 1, 1), (BM, L, 1)) + vm) + molsm
    em = jnp.exp(sm - jnp.max(sm, axis=1, keepdims=True))
    mw = em / jnp.sum(em, axis=1, keepdims=True) * amask3
    afw = jnp.sum(af3 * mw, axis=1)
    mc = _elu(_mm(afw, amt_ref[...]) + hasm * amb_ref[...])
    molf = _gru(mc, molf, mwih, mwhh, mbih, mbhh)
  pred_out[...] = _mm(molf, outw_ref[...]) + outb_ref[0, 0]


def _tc_specs():
  full = lambda shape: pl.BlockSpec(shape, lambda i, _s=len(shape): (0,) * _s)
  in_specs = [
      pl.BlockSpec((R, AF), lambda i: (i, 0)),
      pl.BlockSpec((NPC, R, 128), lambda i: (0, i, 0)),
      pl.BlockSpec((R, D), lambda i: (i, 0)),
      pl.BlockSpec((R, 1), lambda i: (i, 0)),
      full((AF, FP)), full((1, FP)),
      full((NPC, 128, NBW)), full((1, NBW)),
      full((FP, 1)), full((NBW, D)), full((1, 1)),
      full((D, NBW)),
      full((NBW, FP)), full((1, FP)),
      full((FP, FP)), full((1, FP)),
      full((2, 3, FP, FP)), full((2, 3, FP, FP)),
      full((2, 3, 1, FP)), full((2, 3, 1, FP)),
      full((FP, 1)), full((FP, 1)), full((1, 1)),
      full((FP, FP)), full((1, FP)),
      full((3, FP, FP)), full((3, FP, FP)),
      full((3, 1, FP)), full((3, 1, FP)),
      full((FP, 1)), full((1, 1)),
  ]
  out_specs = [
      pl.BlockSpec((R, FP), lambda i: (i, 0)),
      pl.BlockSpec((BM, 1), lambda i: (i, 0)),
  ]
  out_shape = [
      jax.ShapeDtypeStruct((B * L, FP), jnp.float32),
      jax.ShapeDtypeStruct((B, 1), jnp.float32),
  ]
  return (GRID,), in_specs, out_specs, out_shape


def _prep_params(P):
  f32 = lambda x: x.astype(jnp.float32)
  nbW = f32(P['nb_fc_W'])
  wa_t = jnp.pad(nbW[:, :AF].T, ((0, APAD - AF), (0, 0)))
  wb_t = jnp.pad(nbW[:, AF:].T, ((0, BPAD - BF), (0, 0)))
  w2a = f32(P['align_W'])[0, :, FP:].reshape(FP, 1)
  wih = f32(P['gru_Wih']).reshape(2, 3, FP, FP).transpose(0, 1, 3, 2)
  whh = f32(P['gru_Whh']).reshape(2, 3, FP, FP).transpose(0, 1, 3, 2)
  mwih = f32(P['mol_gru_Wih']).reshape(3, FP, FP).transpose(0, 2, 1)
  mwhh = f32(P['mol_gru_Whh']).reshape(3, FP, FP).transpose(0, 2, 1)
  wc = jnp.concatenate([wa_t, wb_t], axis=0)
  return [
      f32(P['atom_fc_W']).T, f32(P['atom_fc_b']).reshape(1, FP),
      block_diag(*([wc] * D)).reshape(NPC, 128, NBW),
      jnp.tile(f32(P['nb_fc_b']).reshape(1, FP), (1, D)),
      f32(P['align_W'])[0, :, :FP].reshape(FP, 1),
      block_diag(*([w2a] * D)),
      f32(P['align_b'])[0].reshape(1, 1),
      jnp.kron(jnp.eye(D, dtype=jnp.float32), jnp.ones((1, FP), jnp.float32)),
      jnp.tile(f32(P['attend_W'])[0].T, (D, 1)),
      f32(P['attend_b'])[0].reshape(1, FP),
      f32(P['attend_W'])[1].T, f32(P['attend_b'])[1].reshape(1, FP),
      wih, whh,
      f32(P['gru_bih']).reshape(2, 3, 1, FP), f32(P['gru_bhh']).reshape(2, 3, 1, FP),
      f32(P['mol_align_W'])[:, :FP].reshape(FP, 1),
      f32(P['mol_align_W'])[:, FP:].reshape(FP, 1),
      f32(P['mol_align_b']).reshape(1, 1),
      f32(P['mol_attend_W']).T, f32(P['mol_attend_b']).reshape(1, FP),
      mwih, mwhh,
      f32(P['mol_gru_bih']).reshape(3, 1, FP), f32(P['mol_gru_bhh']).reshape(3, 1, FP),
      f32(P['out_W']).reshape(FP, 1), f32(P['out_b']).reshape(1, 1),
  ]


def kernel(atom_list, bond_list, atom_degree_list, bond_degree_list,
           atom_mask, params):
  atom2 = atom_list.reshape(B * L, AF).astype(jnp.float32)
  atom_tab = jnp.pad(atom2, ((0, 0), (0, APAD - AF)))
  bond_tab = jnp.pad(bond_list.reshape(B * L, BF).astype(jnp.float32),
                     ((0, 0), (0, BPAD - BF)))
  adl = atom_degree_list.astype(jnp.int32)
  bdl = bond_degree_list.astype(jnp.int32)
  base = (jnp.arange(B, dtype=jnp.int32) * L)[:, None, None]
  ngrp = B * L // CATOM
  gia_t = (adl + base).reshape(ngrp, CATOM, D).transpose(0, 2, 1).reshape(ROWS)
  gib_t = (bdl + base).reshape(ngrp, CATOM, D).transpose(0, 2, 1).reshape(ROWS)

  cnb = _sc_gather(atom_tab, bond_tab, gia_t, gib_t)

  grid, in_specs, out_specs, out_shape = _tc_specs()
  af2, pred = pl.pallas_call(
      _tc_body,
      grid=grid,
      in_specs=in_specs,
      out_specs=out_specs,
      out_shape=out_shape,
      compiler_params=pltpu.CompilerParams(
          dimension_semantics=("arbitrary",),
          vmem_limit_bytes=100 * 1024 * 1024,
      ),
  )(atom2, cnb, adl.reshape(B * L, D),
    atom_mask.reshape(B * L, 1).astype(jnp.float32),
    *_prep_params(params))
  return af2.reshape(B, L, FP), pred


# speedup vs baseline: 1.2900x; 1.0261x over previous
import functools

import jax
import jax.numpy as jnp
from jax import lax
from jax.experimental import pallas as pl
from jax.experimental.pallas import tpu as pltpu
from jax.experimental.pallas import tpu_sc as plsc
from jax.scipy.linalg import block_diag

B, L, D = 512, 48, 6
AF, BF, FP = 39, 10, 64
APAD, BPAD = 48, 16
SLOT = APAD + BPAD
CW = D * SLOT
NBW = D * FP
T = 2

NC, NS = 2, 16
NW = NC * NS
ROWS = B * L * D
RPW = ROWS // NW
CHUNK = 768
NCHUNK = RPW // CHUNK
CATOM = CHUNK // D
NPC = CW // 128

BM = 64
GRID = B // BM
R = BM * L


def _sc_gather(atom_tab, bond_tab, gia_t, gib_t):
  mesh = plsc.VectorSubcoreMesh(core_axis_name="c", subcore_axis_name="s")

  @functools.partial(
      pl.kernel,
      mesh=mesh,
      out_type=jax.ShapeDtypeStruct((NPC, B * L, 128), jnp.float32),
      scratch_types=[
          pltpu.VMEM((NCHUNK, CHUNK), jnp.int32),
          pltpu.VMEM((NCHUNK, CHUNK), jnp.int32),
          pltpu.VMEM((2, CHUNK, APAD), jnp.float32),
          pltpu.VMEM((2, CHUNK, BPAD), jnp.float32),
          pltpu.SemaphoreType.DMA,
          pltpu.SemaphoreType.DMA,
          pltpu.SemaphoreType.DMA,
      ],
      compiler_params=pltpu.CompilerParams(use_tc_tiling_on_sc=False),
  )
  def k(atom_hbm, bond_hbm, ia_hbm, ib_hbm, cnb_hbm,
        ia_v, ib_v, a_v, b_v, sem_i, sem_g, sem_w):
    wid = lax.axis_index("s") * NC + lax.axis_index("c")
    idx_cp = (pltpu.async_copy(
                  ia_hbm.at[pl.ds(wid * NCHUNK, NCHUNK)], ia_v, sem_i),
              pltpu.async_copy(
                  ib_hbm.at[pl.ds(wid * NCHUNK, NCHUNK)], ib_v, sem_i))
    idx_waited = [False]

    def start(c):
      p = c % 2
      if not idx_waited[0]:
        for cp in idx_cp:
          cp.wait()
        idx_waited[0] = True
      return (pltpu.async_copy(atom_hbm.at[ia_v.at[c]], a_v.at[p], sem_g),
              pltpu.async_copy(bond_hbm.at[ib_v.at[c]], b_v.at[p], sem_g))

    pending_w = {0: [], 1: []}
    pending_g = {0: None, 1: None}
    pending_g[0] = start(0)
    for c in range(NCHUNK):
      p = c % 2
      q = (c + 1) % 2
      if c + 1 < NCHUNK:
        for cp in pending_w[q]:
          cp.wait()
        pending_w[q] = []
        pending_g[q] = start(c + 1)
      for cp in pending_g[p]:
        cp.wait()
      aoff = (wid * NCHUNK + c) * CATOM
      ws = []
      for d in range(D):
        lane = (d % 2) * SLOT
        ws.append(pltpu.async_copy(
            a_v.at[p, pl.ds(d * CATOM, CATOM)],
            cnb_hbm.at[d // 2, pl.ds(aoff, CATOM), pl.ds(lane, APAD)], sem_w))
        ws.append(pltpu.async_copy(
            b_v.at[p, pl.ds(d * CATOM, CATOM)],
            cnb_hbm.at[d // 2, pl.ds(aoff, CATOM), pl.ds(lane + APAD, BPAD)],
            sem_w))
      pending_w[p] = ws
    for p in (0, 1):
      for cp in pending_w[p]:
        cp.wait()

  return k(atom_tab, bond_tab, gia_t, gib_t)


def _mm(a, b):
  return lax.dot_general(a, b, (((1,), (0,)), ((), ())),
                         preferred_element_type=jnp.float32)


def _lrelu(x):
  return jnp.where(x >= 0, x, 0.01 * x)


def _elu(x):
  return jnp.where(x > 0, x, jnp.exp(jnp.minimum(x, 0.0)) - 1.0)


def _gru(x, h, wih, whh, bih, bhh):
  g_r = _mm(x, wih[0]) + bih[0] + _mm(h, whh[0]) + bhh[0]
  g_z = _mm(x, wih[1]) + bih[1] + _mm(h, whh[1]) + bhh[1]
  i_n = _mm(x, wih[2]) + bih[2]
  h_n = _mm(h, whh[2]) + bhh[2]
  r = jax.nn.sigmoid(g_r)
  z = jax.nn.sigmoid(g_z)
  n = jnp.tanh(i_n + r * h_n)
  return (1.0 - z) * n + z * h


def _tc_body(atom_ref, cnb_ref, adl_ref, amask_ref,
             wfc_ref, bfc_ref, wcblk_ref, bnbb_ref,
             w1a_ref, w2blk_ref, balign_ref,
             e6_ref, a0stk_ref, a0b_ref, a1t_ref, a1b_ref,
             wih_ref, whh_ref, bih_ref, bhh_ref,
             w1m_ref, w2m_ref, bmal_ref, amt_ref, amb_ref,
             mwih_ref, mwhh_ref, mbih_ref, mbhh_ref,
             outw_ref, outb_ref,
             af_out, pred_out):
  af = _lrelu(_mm(atom_ref[...], wfc_ref[...]) + bfc_ref[...])

  nbf = _lrelu(_mm(cnb_ref[0], wcblk_ref[0]) +
               _mm(cnb_ref[1], wcblk_ref[1]) +
               _mm(cnb_ref[2], wcblk_ref[2]) + bnbb_ref[...])

  adl = adl_ref[...]
  is_pad = adl == (L - 1)
  att = jnp.where(is_pad, 0.0, 1.0)
  smask = jnp.where(is_pad, -9e8, 0.0)
  cnt = jnp.sum(att, axis=1, keepdims=True)
  has = jnp.where(cnt > 0.0, 1.0, 0.0)

  u = _mm(af, w1a_ref[...]) + balign_ref[0, 0]
  v = _mm(nbf, w2blk_ref[...])
  s = _lrelu(u + v) + smask
  e = jnp.exp(s - jnp.max(s, axis=1, keepdims=True))
  aw = e / jnp.sum(e, axis=1, keepdims=True) * att
  awx = _mm(aw, e6_ref[...])
  ctx = _elu(_mm(nbf * awx, a0stk_ref[...]) + has * a0b_ref[...])
  af = _gru(ctx, af,
            (wih_ref[0, 0], wih_ref[0, 1], wih_ref[0, 2]),
            (whh_ref[0, 0], whh_ref[0, 1], whh_ref[0, 2]),
            (bih_ref[0, 0], bih_ref[0, 1], bih_ref[0, 2]),
            (bhh_ref[0, 0], bhh_ref[0, 1], bhh_ref[0, 2]))

  rfeat = jnp.maximum(af, 0.0)
  ctx1 = _elu(has * (_mm(rfeat, a1t_ref[...]) + a1b_ref[...]))
  af = _gru(ctx1, af,
            (wih_ref[1, 0], wih_ref[1, 1], wih_ref[1, 2]),
            (whh_ref[1, 0], whh_ref[1, 1], whh_ref[1, 2]),
            (bih_ref[1, 0], bih_ref[1, 1], bih_ref[1, 2]),
            (bhh_ref[1, 0], bhh_ref[1, 1], bhh_ref[1, 2]))
  af_out[...] = af

  amask = amask_ref[...]
  molf = jnp.sum((jnp.maximum(af, 0.0) * amask).reshape(BM, L, FP), axis=1)
  molsm = jnp.where(amask == 0.0, -9e8, 0.0).reshape(BM, L, 1)
  amask3 = amask.reshape(BM, L, 1)
  hasm = jnp.where(jnp.sum(amask3, axis=1) > 0.0, 1.0, 0.0)
  mwih = (mwih_ref[0], mwih_ref[1], mwih_ref[2])
  mwhh = (mwhh_ref[0], mwhh_ref[1], mwhh_ref[2])
  mbih = (mbih_ref[0], mbih_ref[1], mbih_ref[2])
  mbhh = (mbhh_ref[0], mbhh_ref[1], mbhh_ref[2])
  af3 = af.reshape(BM, L, FP)
  vm = (_mm(af, w2m_ref[...]) + bmal_ref[0, 0]).reshape(BM, L, 1)
  for _ in range(T):
    um = _mm(molf, w1m_ref[...])
    sm = _lrelu(jnp.broadcast_to(um.reshape(BM, 1, 1), (BM, L, 1)) + vm) + molsm
    em = jnp.exp(sm - jnp.max(sm, axis=1, keepdims=True))
    mw = em / jnp.sum(em, axis=1, keepdims=True) * amask3
    afw = jnp.sum(af3 * mw, axis=1)
    mc = _elu(_mm(afw, amt_ref[...]) + hasm * amb_ref[...])
    molf = _gru(mc, molf, mwih, mwhh, mbih, mbhh)
  pred_out[...] = _mm(molf, outw_ref[...]) + outb_ref[0, 0]


def _tc_specs():
  full = lambda shape: pl.BlockSpec(shape, lambda i, _s=len(shape): (0,) * _s)
  in_specs = [
      pl.BlockSpec((R, AF), lambda i: (i, 0)),
      pl.BlockSpec((NPC, R, 128), lambda i: (0, i, 0)),
      pl.BlockSpec((R, D), lambda i: (i, 0)),
      pl.BlockSpec((R, 1), lambda i: (i, 0)),
      full((AF, FP)), full((1, FP)),
      full((NPC, 128, NBW)), full((1, NBW)),
      full((FP, 1)), full((NBW, D)), full((1, 1)),
      full((D, NBW)),
      full((NBW, FP)), full((1, FP)),
      full((FP, FP)), full((1, FP)),
      full((2, 3, FP, FP)), full((2, 3, FP, FP)),
      full((2, 3, 1, FP)), full((2, 3, 1, FP)),
      full((FP, 1)), full((FP, 1)), full((1, 1)),
      full((FP, FP)), full((1, FP)),
      full((3, FP, FP)), full((3, FP, FP)),
      full((3, 1, FP)), full((3, 1, FP)),
      full((FP, 1)), full((1, 1)),
  ]
  out_specs = [
      pl.BlockSpec((R, FP), lambda i: (i, 0)),
      pl.BlockSpec((BM, 1), lambda i: (i, 0)),
  ]
  out_shape = [
      jax.ShapeDtypeStruct((B * L, FP), jnp.float32),
      jax.ShapeDtypeStruct((B, 1), jnp.float32),
  ]
  return (GRID,), in_specs, out_specs, out_shape


def _prep_params(P):
  f32 = lambda x: x.astype(jnp.float32)
  nbW = f32(P['nb_fc_W'])
  wa_t = jnp.pad(nbW[:, :AF].T, ((0, APAD - AF), (0, 0)))
  wb_t = jnp.pad(nbW[:, AF:].T, ((0, BPAD - BF), (0, 0)))
  w2a = f32(P['align_W'])[0, :, FP:].reshape(FP, 1)
  wih = f32(P['gru_Wih']).reshape(2, 3, FP, FP).transpose(0, 1, 3, 2)
  whh = f32(P['gru_Whh']).reshape(2, 3, FP, FP).transpose(0, 1, 3, 2)
  mwih = f32(P['mol_gru_Wih']).reshape(3, FP, FP).transpose(0, 2, 1)
  mwhh = f32(P['mol_gru_Whh']).reshape(3, FP, FP).transpose(0, 2, 1)
  wc = jnp.concatenate([wa_t, wb_t], axis=0)
  return [
      f32(P['atom_fc_W']).T, f32(P['atom_fc_b']).reshape(1, FP),
      block_diag(*([wc] * D)).reshape(NPC, 128, NBW),
      jnp.tile(f32(P['nb_fc_b']).reshape(1, FP), (1, D)),
      f32(P['align_W'])[0, :, :FP].reshape(FP, 1),
      block_diag(*([w2a] * D)),
      f32(P['align_b'])[0].reshape(1, 1),
      jnp.kron(jnp.eye(D, dtype=jnp.float32), jnp.ones((1, FP), jnp.float32)),
      jnp.tile(f32(P['attend_W'])[0].T, (D, 1)),
      f32(P['attend_b'])[0].reshape(1, FP),
      f32(P['attend_W'])[1].T, f32(P['attend_b'])[1].reshape(1, FP),
      wih, whh,
      f32(P['gru_bih']).reshape(2, 3, 1, FP), f32(P['gru_bhh']).reshape(2, 3, 1, FP),
      f32(P['mol_align_W'])[:, :FP].reshape(FP, 1),
      f32(P['mol_align_W'])[:, FP:].reshape(FP, 1),
      f32(P['mol_align_b']).reshape(1, 1),
      f32(P['mol_attend_W']).T, f32(P['mol_attend_b']).reshape(1, FP),
      mwih, mwhh,
      f32(P['mol_gru_bih']).reshape(3, 1, FP), f32(P['mol_gru_bhh']).reshape(3, 1, FP),
      f32(P['out_W']).reshape(FP, 1), f32(P['out_b']).reshape(1, 1),
  ]


def kernel(atom_list, bond_list, atom_degree_list, bond_degree_list,
           atom_mask, params):
  atom2 = atom_list.reshape(B * L, AF).astype(jnp.float32)
  atom_tab = jnp.pad(atom2, ((0, 0), (0, APAD - AF)))
  bond_tab = jnp.pad(bond_list.reshape(B * L, BF).astype(jnp.float32),
                     ((0, 0), (0, BPAD - BF)))
  adl = atom_degree_list.astype(jnp.int32)
  bdl = bond_degree_list.astype(jnp.int32)
  base = (jnp.arange(B, dtype=jnp.int32) * L)[:, None, None]
  ngrp = B * L // CATOM
  gia_t = (adl + base).reshape(ngrp, CATOM, D).transpose(0, 2, 1).reshape(ngrp, CHUNK)
  gib_t = (bdl + base).reshape(ngrp, CATOM, D).transpose(0, 2, 1).reshape(ngrp, CHUNK)

  cnb = _sc_gather(atom_tab, bond_tab, gia_t, gib_t)

  grid, in_specs, out_specs, out_shape = _tc_specs()
  af2, pred = pl.pallas_call(
      _tc_body,
      grid=grid,
      in_specs=in_specs,
      out_specs=out_specs,
      out_shape=out_shape,
      compiler_params=pltpu.CompilerParams(
          dimension_semantics=("arbitrary",),
          vmem_limit_bytes=100 * 1024 * 1024,
      ),
  )(atom2, cnb, adl.reshape(B * L, D),
    atom_mask.reshape(B * L, 1).astype(jnp.float32),
    *_prep_params(params))
  return af2.reshape(B, L, FP), pred
